# baseline (speedup 1.0000x reference)
import functools

import jax
import jax.numpy as jnp
from jax import lax
from jax.experimental import pallas as pl
from jax.experimental.pallas import tpu as pltpu
from jax.experimental.pallas import tpu_sc as plsc

F = 26
B = 4096
H = 20
V = 100000
D = 32
L = 16

NW = 32
PAIRS_PER_TILE = (F * D) // NW


def _make_sc_kernel():
    info = plsc.get_sparse_core_info()
    nc = info.num_cores
    mesh = plsc.VectorSubcoreMesh(core_axis_name="c", subcore_axis_name="s")

    @functools.partial(
        pl.kernel,
        mesh=mesh,
        compiler_params=pltpu.CompilerParams(needs_layout_passes=False),
        out_type=jax.ShapeDtypeStruct((F, D, B), jnp.float32),
        scratch_types=[
            pltpu.VMEM((V,), jnp.float32),
            pltpu.VMEM((B,), jnp.int32),
            pltpu.VMEM((B,), jnp.int32),
            pltpu.VMEM((B,), jnp.float32),
            pltpu.SemaphoreType.DMA,
        ],
    )
    def k(tab_hbm, idx_hbm, out_hbm, slab_v, idx_a, idx_b, acc_v, sem):
        wid = lax.axis_index("s") * nc + lax.axis_index("c")

        def pair_body(i, carry):
            p = wid * PAIRS_PER_TILE + i
            f = lax.shift_right_logical(p, 5)
            d = lax.bitwise_and(p, D - 1)
            pltpu.sync_copy(tab_hbm.at[f, d], slab_v)
            pltpu.sync_copy(idx_hbm.at[f, 0], idx_a)
            bufs = (idx_a, idx_b)
            for l in range(H):
                cur = bufs[l % 2]
                nxt = bufs[(l + 1) % 2]
                if l + 1 < H:
                    cp = pltpu.async_copy(idx_hbm.at[f, l + 1], nxt, sem)

                first = l == 0

                @plsc.parallel_loop(0, B // L, unroll=16)
                def j_body(j):
                    iv = cur[pl.ds(j * L, L)]
                    g = plsc.load_gather(slab_v, [iv])
                    if first:
                        acc_v[pl.ds(j * L, L)] = g
                    else:
                        plsc.addupdate(acc_v.at[pl.ds(j * L, L)], g)
                if l + 1 < H:
                    cp.wait()
            pltpu.sync_copy(acc_v, out_hbm.at[f, d])
            return carry

        lax.fori_loop(0, PAIRS_PER_TILE, pair_body, 0)

    return k


_sc_kernel = _make_sc_kernel()


@jax.jit
def kernel(inputs, tables):
    idx_t = jnp.transpose(inputs.astype(jnp.int32), (0, 2, 1))
    tab_t = jnp.transpose(tables, (0, 2, 1))
    out = _sc_kernel(tab_t, idx_t)
    return jnp.transpose(out, (2, 0, 1))


# speedup vs baseline: 1.2961x; 1.2961x over previous
import functools

import jax
import jax.numpy as jnp
from jax import lax
from jax.experimental import pallas as pl
from jax.experimental.pallas import tpu as pltpu
from jax.experimental.pallas import tpu_sc as plsc

F = 26
B = 4096
H = 20
V = 100000
D = 32
L = 16

NW = 32
PAIRS_PER_TILE = (F * D) // NW


def _make_sc_kernel():
    info = plsc.get_sparse_core_info()
    nc = info.num_cores
    mesh = plsc.VectorSubcoreMesh(core_axis_name="c", subcore_axis_name="s")

    @functools.partial(
        pl.kernel,
        mesh=mesh,
        compiler_params=pltpu.CompilerParams(needs_layout_passes=False),
        out_type=jax.ShapeDtypeStruct((F, D, B), jnp.float32),
        scratch_types=[
            pltpu.VMEM((V,), jnp.float32),
            pltpu.VMEM((B,), jnp.int32),
            pltpu.VMEM((B,), jnp.int32),
            pltpu.VMEM((B,), jnp.int32),
            pltpu.VMEM((B,), jnp.int32),
            pltpu.VMEM((B,), jnp.float32),
            pltpu.SemaphoreType.DMA,
        ],
    )
    def k(tab_hbm, idx_hbm, out_hbm, slab_v, ia0, ia1, ib0, ib1, acc_v, sem):
        wid = lax.axis_index("s") * nc + lax.axis_index("c")
        T = H // 2

        def pair_body(i, carry):
            p = wid * PAIRS_PER_TILE + i
            f = lax.shift_right_logical(p, 5)
            d = lax.bitwise_and(p, D - 1)
            pltpu.sync_copy(tab_hbm.at[f, d], slab_v)
            pltpu.sync_copy(idx_hbm.at[f, 0], ia0)
            pltpu.sync_copy(idx_hbm.at[f, 1], ia1)
            bufs = ((ia0, ia1), (ib0, ib1))
            for t in range(T):
                cur0, cur1 = bufs[t % 2]
                nxt0, nxt1 = bufs[(t + 1) % 2]
                if t + 1 < T:
                    cp0 = pltpu.async_copy(idx_hbm.at[f, 2 * t + 2], nxt0, sem)
                    cp1 = pltpu.async_copy(idx_hbm.at[f, 2 * t + 3], nxt1, sem)

                first = t == 0

                @plsc.parallel_loop(0, B // L, unroll=8)
                def j_body(j):
                    sl = pl.ds(j * L, L)
                    g = plsc.load_gather(slab_v, [cur0[sl]]) + plsc.load_gather(
                        slab_v, [cur1[sl]]
                    )
                    if first:
                        acc_v[sl] = g
                    else:
                        plsc.addupdate(acc_v.at[sl], g)
                if t + 1 < T:
                    cp0.wait()
                    cp1.wait()
            pltpu.sync_copy(acc_v, out_hbm.at[f, d])
            return carry

        lax.fori_loop(0, PAIRS_PER_TILE, pair_body, 0)

    return k


_sc_kernel = _make_sc_kernel()


@jax.jit
def kernel(inputs, tables):
    idx_t = jnp.transpose(inputs.astype(jnp.int32), (0, 2, 1))
    tab_t = jnp.transpose(tables, (0, 2, 1))
    out = _sc_kernel(tab_t, idx_t)
    return jnp.transpose(out, (2, 0, 1))
